# baseline (speedup 1.0000x reference)
import jax
import jax.numpy as jnp
from jax.experimental import pallas as pl
from jax.experimental.pallas import tpu as pltpu


def _copy_body(pe_hbm, o_hbm, sem):
    copy = pltpu.make_async_copy(pe_hbm, o_hbm, sem)
    copy.start()
    copy.wait()


def kernel(idx, pe):
    T = idx.shape[1]
    out = pl.pallas_call(
        _copy_body,
        out_shape=jax.ShapeDtypeStruct((T, pe.shape[1]), pe.dtype),
        in_specs=[pl.BlockSpec(memory_space=pl.ANY)],
        out_specs=pl.BlockSpec(memory_space=pl.ANY),
        scratch_shapes=[pltpu.SemaphoreType.DMA],
    )(pe)
    return out[None, :, :]


# speedup vs baseline: 3.4451x; 3.4451x over previous
import jax
import jax.numpy as jnp
from jax.experimental import pallas as pl
from jax.experimental.pallas import tpu as pltpu

_GRID = 8


def _copy_body(pe_ref, o_ref):
    o_ref[...] = pe_ref[...]


def kernel(idx, pe):
    T = idx.shape[1]
    D = pe.shape[1]
    rows = T // _GRID
    out = pl.pallas_call(
        _copy_body,
        grid=(_GRID,),
        out_shape=jax.ShapeDtypeStruct((T, D), pe.dtype),
        in_specs=[pl.BlockSpec((rows, D), lambda i: (i, 0))],
        out_specs=pl.BlockSpec((rows, D), lambda i: (i, 0)),
    )(pe)
    return out[None, :, :]


# speedup vs baseline: 8.9989x; 2.6121x over previous
import jax
import jax.numpy as jnp
from jax.experimental import pallas as pl
from jax.experimental.pallas import tpu as pltpu

_GRID = 2


def _copy_body(pe_ref, o_ref):
    o_ref[...] = pe_ref[...]


def kernel(idx, pe):
    T = idx.shape[1]
    D = pe.shape[1]
    rows = T // _GRID
    out = pl.pallas_call(
        _copy_body,
        grid=(_GRID,),
        out_shape=jax.ShapeDtypeStruct((T, D), pe.dtype),
        in_specs=[pl.BlockSpec((rows, D), lambda i: (i, 0))],
        out_specs=pl.BlockSpec((rows, D), lambda i: (i, 0)),
    )(pe)
    return out[None, :, :]
